# Initial kernel scaffold; baseline (speedup 1.0000x reference)
import jax
import jax.numpy as jnp
from jax.experimental import pallas as pl


def kernel(origins, directions, weights, bins, max_bin):
    raise NotImplementedError("write your pallas kernel here")


# speedup vs baseline: 8.5544x; 8.5544x over previous
import functools

import jax
import jax.numpy as jnp
from jax import lax
from jax.experimental import pallas as pl
from jax.experimental.pallas import tpu as pltpu
from jax.experimental.pallas import tpu_sc as plsc

_R = 131072
_NC = 64
_NB = 129
_NK = _NC + 1
_TL = _NK + _NB
_ROWS = _TL + 1
_OB = _NC + _NB - 1
_PAD = 0.01
_EPS = 1e-5
_NCORES = 2
_NSUB = 16
_NW = _NCORES * _NSUB
_G = 16
_RPW = _R // _NW
_CHUNKS = _RPW // _G
_OSZ = _G * _OB
_OBUF = _OSZ + 128


def _sc_sample(wf, bf, mb1d):
    mesh = plsc.VectorSubcoreMesh(core_axis_name="c", subcore_axis_name="s")

    @functools.partial(
        pl.kernel,
        out_type=[
            jax.ShapeDtypeStruct((_R * _OB,), jnp.float32),
            jax.ShapeDtypeStruct((_R,), jnp.float32),
        ],
        mesh=mesh,
        compiler_params=pltpu.CompilerParams(needs_layout_passes=False),
        scratch_types=[
            pltpu.VMEM((_G * _NC,), jnp.float32),
            pltpu.VMEM((_G * _NC,), jnp.float32),
            pltpu.VMEM((_G,), jnp.float32),
            pltpu.VMEM((_ROWS * _G,), jnp.float32),
            pltpu.VMEM((_ROWS * _G,), jnp.float32),
            pltpu.VMEM((_NK * _G,), jnp.int32),
            pltpu.VMEM((_OBUF,), jnp.float32),
        ],
    )
    def kern(w_hbm, b_hbm, mb_hbm, ob_hbm, omb_hbm,
             wbuf, bbuf, mbuf, dA, dB, cbuf, obuf):
        wid = lax.axis_index("s") * _NCORES + lax.axis_index("c")
        base0 = wid * _RPW
        lane = lax.iota(jnp.int32, _G)
        zf = jnp.zeros((_G,), jnp.float32)
        zi = jnp.zeros((_G,), jnp.int32)
        oi = jnp.ones((_G,), jnp.int32)
        row = lane * _OB
        grow = lane * _NC

        def zinit(t, carry):
            dA[pl.ds(t * _G, _G)] = zf
            dB[pl.ds(t * _G, _G)] = zf
            return carry

        lax.fori_loop(0, _ROWS, zinit, 0)

        def col(k):
            return jnp.full((_G,), k, jnp.int32)

        def oidx(pos):
            return jnp.where(pos < _OB, row + pos,
                             _OSZ + lane + (pos - _OB) * 32)

        def chunk(ci, carry):
            base = base0 + ci * _G
            pltpu.sync_copy(w_hbm.at[pl.ds(base * _NC, _G * _NC)], wbuf)
            pltpu.sync_copy(b_hbm.at[pl.ds(base * _NC, _G * _NC)], bbuf)
            pltpu.sync_copy(mb_hbm.at[pl.ds(base, _G)], mbuf)

            def sbody(k, s):
                return s + plsc.load_gather(wbuf, [grow + k])

            S = lax.fori_loop(0, _NC, sbody, zf) + (_NC * _PAD)
            padding = jnp.maximum(_EPS - S, 0.0)
            rinv = 1.0 / (S + padding)
            padc = padding * (1.0 / _NC) + _PAD

            e0 = plsc.load_gather(bbuf, [grow])
            cbuf[pl.ds(0, _G)] = zi

            def seg_step(k, ek, carry):
                cs, cdfp, cp, ep, Ap, Bp = carry
                kf = jnp.asarray(k, jnp.int32).astype(jnp.float32)
                wk = plsc.load_gather(wbuf, [grow + (k - 1)])
                cs = cs + (wk + padc) * rinv
                cdfk = jnp.minimum(cs, 1.0)
                y = cdfk * _NB - 0.5
                iy = y.astype(jnp.int32)
                ck = jnp.maximum(
                    iy + jnp.where(iy.astype(jnp.float32) < y, oi, zi), 0)
                cbuf[pl.ds(k * _G, _G)] = ck
                d = cdfk - cdfp
                inv = jnp.where(d > 0, 1.0 / d, 0.0)
                g = (ek - ep) * inv
                beta = g * (1.0 / _NB)
                Am = ep + g * ((0.5 - kf) * (1.0 / _NB) - cdfp)
                idx = (cp + k) * _G + lane
                plsc.addupdate_scatter(dA, [idx], Am - Ap)
                plsc.addupdate_scatter(dB, [idx], beta - Bp)
                return (cs, cdfk, ck, ek, Am, beta)

            def kbody(k, carry):
                ek = plsc.load_gather(bbuf, [grow + k])
                return seg_step(k, ek, carry)

            carry0 = (zf, zf, zi, e0, zf, zf)
            carry1 = lax.fori_loop(1, _NC, kbody, carry0)
            mvec = mbuf[...]
            cs, cdfk, ck, ek, Am, Bm = seg_step(_NC, mvec, carry1)
            idx = (ck + (_NC + 1)) * _G + lane
            plsc.addupdate_scatter(dA, [idx], mvec - Am)
            plsc.addupdate_scatter(dB, [idx], -Bm)

            def tbody(t, carry):
                cA, cB = carry
                off = t * _G
                a = dA[pl.ds(off, _G)]
                dA[pl.ds(off, _G)] = zf
                b = dB[pl.ds(off, _G)]
                dB[pl.ds(off, _G)] = zf
                cA = cA + a
                cB = cB + b
                tf = jnp.asarray(t, jnp.int32).astype(jnp.float32)
                plsc.store_scatter(obuf, [oidx(col(t))], cA + cB * tf)
                return (cA, cB)

            lax.fori_loop(0, _ROWS, tbody, (zf, zf))

            def obody(k, carry):
                ckk = cbuf[pl.ds(k * _G, _G)]
                ekk = plsc.load_gather(bbuf, [grow + k])
                plsc.store_scatter(obuf, [oidx(ckk + k)], ekk)
                return carry

            lax.fori_loop(0, _NC, obody, 0)
            cklast = cbuf[pl.ds(_NC * _G, _G)]
            plsc.store_scatter(obuf, [oidx(cklast + _NC)], mvec)

            pltpu.sync_copy(obuf.at[pl.ds(0, _OSZ)],
                            ob_hbm.at[pl.ds(base * _OB, _OSZ)])
            pltpu.sync_copy(obuf.at[pl.ds(_OSZ, _G)],
                            omb_hbm.at[pl.ds(base, _G)])
            return carry

        lax.fori_loop(0, _CHUNKS, chunk, 0)

    return kern(wf, bf, mb1d)


def kernel(origins, directions, weights, bins, max_bin):
    ob, omb = _sc_sample(weights.reshape(_R * _NC),
                         bins.reshape(_R * _NC), max_bin[:, 0])
    return (origins, directions, ob.reshape(_R, _OB, 1), omb[:, None])


# speedup vs baseline: 10.0377x; 1.1734x over previous
import functools

import jax
import jax.numpy as jnp
from jax import lax
from jax.experimental import pallas as pl
from jax.experimental.pallas import tpu as pltpu
from jax.experimental.pallas import tpu_sc as plsc

_R = 131072
_NC = 64
_NB = 129
_NK = _NC + 1
_OB = _NC + _NB - 1
_ROWS = 194
_PAD = 0.01
_EPS = 1e-5
_NCORES = 2
_NSUB = 16
_NW = _NCORES * _NSUB
_L = 16
_NG = 2
_G = _NG * _L
_RPW = _R // _NW
_CHUNKS = _RPW // _G
_OSZ = _G * _OB


def _sc_sample(wf, bf, mb1d):
    mesh = plsc.VectorSubcoreMesh(core_axis_name="c", subcore_axis_name="s")

    @functools.partial(
        pl.kernel,
        out_type=[
            jax.ShapeDtypeStruct((_R * _OB,), jnp.float32),
            jax.ShapeDtypeStruct((_R,), jnp.float32),
        ],
        mesh=mesh,
        compiler_params=pltpu.CompilerParams(needs_layout_passes=False),
        scratch_types=[
            pltpu.VMEM((_G * _NC,), jnp.float32),
            pltpu.VMEM((_G * _NC,), jnp.float32),
            pltpu.VMEM((_G,), jnp.float32),
            pltpu.VMEM((_ROWS * _G,), jnp.float32),
            pltpu.VMEM((_ROWS * _G,), jnp.float32),
            pltpu.VMEM((_OSZ,), jnp.float32),
            pltpu.VMEM((_G,), jnp.float32),
        ],
    )
    def kern(w_hbm, b_hbm, mb_hbm, ob_hbm, omb_hbm,
             wbuf, bbuf, mbuf, dV, dB, obuf, mstage):
        wid = lax.axis_index("s") * _NCORES + lax.axis_index("c")
        base0 = wid * _RPW
        lane = lax.iota(jnp.int32, _L)
        zf = jnp.zeros((_L,), jnp.float32)
        zi = jnp.zeros((_L,), jnp.int32)
        oi = jnp.ones((_L,), jnp.int32)
        glv = [lane + g * _L for g in range(_NG)]
        grow = [(g * _L) * _NC for g in range(_NG)]
        obase = [(jnp.asarray(g * _L, jnp.int32) + lane) * _OB
                 for g in range(_NG)]

        def zinit(t, carry):
            for g in range(_NG):
                dV[pl.ds(t * _G + g * _L, _L)] = zf
                dB[pl.ds(t * _G + g * _L, _L)] = zf
            return carry

        lax.fori_loop(0, _ROWS, zinit, 0)

        def chunk(ci, carry):
            base = base0 + ci * _G
            pltpu.sync_copy(w_hbm.at[pl.ds(base * _NC, _G * _NC)], wbuf)
            pltpu.sync_copy(b_hbm.at[pl.ds(base * _NC, _G * _NC)], bbuf)
            pltpu.sync_copy(mb_hbm.at[pl.ds(base, _G)], mbuf)

            def sbody(k, ss):
                return tuple(
                    ss[g] + plsc.load_gather(wbuf, [glv[g] * _NC + k])
                    for g in range(_NG))

            S = lax.fori_loop(0, _NC, sbody, (zf,) * _NG)
            rinv, padc = [], []
            for g in range(_NG):
                Sg = S[g] + (_NC * _PAD)
                padding = jnp.maximum(_EPS - Sg, 0.0)
                rinv.append(1.0 / (Sg + padding))
                padc.append(padding * (1.0 / _NC) + _PAD)

            mvec = [mbuf[pl.ds(g * _L, _L)] for g in range(_NG)]
            e0 = [plsc.load_gather(bbuf, [glv[g] * _NC]) for g in range(_NG)]
            for g in range(_NG):
                plsc.store_scatter(dV, [glv[g]], e0[g])

            def seg_step(k, ek, st, g):
                cs, cdfp, ep, tpf, idxp = st
                wk = plsc.load_gather(wbuf, [glv[g] * _NC + (k - 1)])
                cs = cs + (wk + padc[g]) * rinv[g]
                cdfk = jnp.minimum(cs, 1.0)
                y = cdfk * _NB - 0.5
                iy = y.astype(jnp.int32)
                ck = jnp.maximum(
                    iy + jnp.where(iy.astype(jnp.float32) < y, oi, zi), 0)
                d = cdfk - cdfp
                inv = jnp.where(d > 0, 1.0 / d, 0.0)
                gr = (ek - ep) * inv
                beta = gr * (1.0 / _NB)
                Am = ep + gr * ((0.5 - k) * (1.0 / _NB) - cdfp)
                dv0 = Am + beta * tpf - ep
                idx0 = idxp + _G
                plsc.addupdate_scatter(dV, [idx0], dv0)
                plsc.addupdate_scatter(dB, [idx0], beta)
                tE = ck + k
                tEf = tE.astype(jnp.float32)
                dvE = ek - (Am + beta * (tEf - 1.0))
                idxE = tE * _G + glv[g]
                plsc.addupdate_scatter(dV, [idxE], dvE)
                plsc.addupdate_scatter(dB, [idxE], -beta)
                return (cs, cdfk, ek, tEf, idxE)

            def kbody(k, sts):
                return tuple(
                    seg_step(k, plsc.load_gather(bbuf, [glv[g] * _NC + k]),
                             sts[g], g)
                    for g in range(_NG))

            sts0 = tuple((zf, zf, e0[g], zf, glv[g]) for g in range(_NG))
            sts = lax.fori_loop(1, _NC, kbody, sts0)
            for g in range(_NG):
                seg_step(_NC, mvec[g], sts[g], g)

            def tbody(t, st):
                off = t * _G
                out = []
                for g in range(_NG):
                    cB, val, oidx = st[g]
                    a = dV[pl.ds(off + g * _L, _L)]
                    dV[pl.ds(off + g * _L, _L)] = zf
                    b = dB[pl.ds(off + g * _L, _L)]
                    dB[pl.ds(off + g * _L, _L)] = zf
                    cB = cB + b
                    val = val + cB + a
                    plsc.store_scatter(obuf, [oidx], val)
                    out.append((cB, val, oidx + 1))
                return tuple(out)

            st0 = tuple((zf, zf, obase[g]) for g in range(_NG))
            st = lax.fori_loop(0, _OB, tbody, st0)
            off = _OB * _G
            for g in range(_NG):
                cB, val, _ = st[g]
                a = dV[pl.ds(off + g * _L, _L)]
                dV[pl.ds(off + g * _L, _L)] = zf
                b = dB[pl.ds(off + g * _L, _L)]
                dB[pl.ds(off + g * _L, _L)] = zf
                mstage[pl.ds(g * _L, _L)] = val + (cB + b) + a
                dV[pl.ds(off + _G + g * _L, _L)] = zf
                dB[pl.ds(off + _G + g * _L, _L)] = zf

            pltpu.sync_copy(obuf, ob_hbm.at[pl.ds(base * _OB, _OSZ)])
            pltpu.sync_copy(mstage, omb_hbm.at[pl.ds(base, _G)])
            return carry

        lax.fori_loop(0, _CHUNKS, chunk, 0)

    return kern(wf, bf, mb1d)


def kernel(origins, directions, weights, bins, max_bin):
    ob, omb = _sc_sample(weights.reshape(_R * _NC),
                         bins.reshape(_R * _NC), max_bin[:, 0])
    return (origins, directions, ob.reshape(_R, _OB, 1), omb[:, None])


# speedup vs baseline: 10.5791x; 1.0539x over previous
import functools

import jax
import jax.numpy as jnp
from jax import lax
from jax.experimental import pallas as pl
from jax.experimental.pallas import tpu as pltpu
from jax.experimental.pallas import tpu_sc as plsc

_R = 131072
_NC = 64
_NB = 129
_NK = _NC + 1
_OB = _NC + _NB - 1
_ROWS = 194
_PAD = 0.01
_EPS = 1e-5
_NCORES = 2
_NSUB = 16
_NW = _NCORES * _NSUB
_L = 16
_NG = 4
_G = _NG * _L
_RPW = _R // _NW
_CHUNKS = _RPW // _G
_OSZ = _G * _OB


def _sc_sample(wf, bf, mb1d):
    mesh = plsc.VectorSubcoreMesh(core_axis_name="c", subcore_axis_name="s")

    @functools.partial(
        pl.kernel,
        out_type=[
            jax.ShapeDtypeStruct((_R * _OB,), jnp.float32),
            jax.ShapeDtypeStruct((_R,), jnp.float32),
        ],
        mesh=mesh,
        compiler_params=pltpu.CompilerParams(needs_layout_passes=False),
        scratch_types=[
            pltpu.VMEM((_G * _NC,), jnp.float32),
            pltpu.VMEM((_G * _NC,), jnp.float32),
            pltpu.VMEM((_G,), jnp.float32),
            pltpu.VMEM((_ROWS * _G,), jnp.float32),
            pltpu.VMEM((_ROWS * _G,), jnp.float32),
            pltpu.VMEM((_OSZ,), jnp.float32),
            pltpu.VMEM((_G,), jnp.float32),
        ],
    )
    def kern(w_hbm, b_hbm, mb_hbm, ob_hbm, omb_hbm,
             wbuf, bbuf, mbuf, dV, dB, obuf, mstage):
        wid = lax.axis_index("s") * _NCORES + lax.axis_index("c")
        base0 = wid * _RPW
        lane = lax.iota(jnp.int32, _L)
        zf = jnp.zeros((_L,), jnp.float32)
        zi = jnp.zeros((_L,), jnp.int32)
        oi = jnp.ones((_L,), jnp.int32)
        glv = [lane + g * _L for g in range(_NG)]
        grow = [(g * _L) * _NC for g in range(_NG)]
        obase = [(jnp.asarray(g * _L, jnp.int32) + lane) * _OB
                 for g in range(_NG)]

        def zinit(t, carry):
            for g in range(_NG):
                dV[pl.ds(t * _G + g * _L, _L)] = zf
                dB[pl.ds(t * _G + g * _L, _L)] = zf
            return carry

        lax.fori_loop(0, _ROWS, zinit, 0)

        def chunk(ci, carry):
            base = base0 + ci * _G
            pltpu.sync_copy(w_hbm.at[pl.ds(base * _NC, _G * _NC)], wbuf)
            pltpu.sync_copy(b_hbm.at[pl.ds(base * _NC, _G * _NC)], bbuf)
            pltpu.sync_copy(mb_hbm.at[pl.ds(base, _G)], mbuf)

            def sbody(k, ss):
                return tuple(
                    ss[g] + plsc.load_gather(wbuf, [glv[g] * _NC + k])
                    for g in range(_NG))

            S = lax.fori_loop(0, _NC, sbody, (zf,) * _NG)
            rinv, padc = [], []
            for g in range(_NG):
                Sg = S[g] + (_NC * _PAD)
                padding = jnp.maximum(_EPS - Sg, 0.0)
                rinv.append(1.0 / (Sg + padding))
                padc.append(padding * (1.0 / _NC) + _PAD)

            mvec = [mbuf[pl.ds(g * _L, _L)] for g in range(_NG)]
            e0 = [plsc.load_gather(bbuf, [glv[g] * _NC]) for g in range(_NG)]
            for g in range(_NG):
                plsc.store_scatter(dV, [glv[g]], e0[g])

            def seg_step(k, ek, st, g):
                cs, cdfp, ep, tpf, idxp = st
                wk = plsc.load_gather(wbuf, [glv[g] * _NC + (k - 1)])
                cs = cs + (wk + padc[g]) * rinv[g]
                cdfk = jnp.minimum(cs, 1.0)
                y = cdfk * _NB - 0.5
                iy = y.astype(jnp.int32)
                ck = jnp.maximum(
                    iy + jnp.where(iy.astype(jnp.float32) < y, oi, zi), 0)
                d = cdfk - cdfp
                inv = jnp.where(d > 0, 1.0 / d, 0.0)
                gr = (ek - ep) * inv
                beta = gr * (1.0 / _NB)
                Am = ep + gr * ((0.5 - k) * (1.0 / _NB) - cdfp)
                dv0 = Am + beta * tpf - ep
                idx0 = idxp + _G
                plsc.addupdate_scatter(dV, [idx0], dv0)
                plsc.addupdate_scatter(dB, [idx0], beta)
                tE = ck + k
                tEf = tE.astype(jnp.float32)
                dvE = ek - (Am + beta * (tEf - 1.0))
                idxE = tE * _G + glv[g]
                plsc.addupdate_scatter(dV, [idxE], dvE)
                plsc.addupdate_scatter(dB, [idxE], -beta)
                return (cs, cdfk, ek, tEf, idxE)

            def kbody(k, sts):
                return tuple(
                    seg_step(k, plsc.load_gather(bbuf, [glv[g] * _NC + k]),
                             sts[g], g)
                    for g in range(_NG))

            sts0 = tuple((zf, zf, e0[g], zf, glv[g]) for g in range(_NG))
            sts = lax.fori_loop(1, _NC, kbody, sts0)
            for g in range(_NG):
                seg_step(_NC, mvec[g], sts[g], g)

            def tbody(t, st):
                off = t * _G
                out = []
                for g in range(_NG):
                    cB, val, oidx = st[g]
                    a = dV[pl.ds(off + g * _L, _L)]
                    dV[pl.ds(off + g * _L, _L)] = zf
                    b = dB[pl.ds(off + g * _L, _L)]
                    dB[pl.ds(off + g * _L, _L)] = zf
                    cB = cB + b
                    val = val + cB + a
                    plsc.store_scatter(obuf, [oidx], val)
                    out.append((cB, val, oidx + 1))
                return tuple(out)

            st0 = tuple((zf, zf, obase[g]) for g in range(_NG))
            st = lax.fori_loop(0, _OB, tbody, st0)
            off = _OB * _G
            for g in range(_NG):
                cB, val, _ = st[g]
                a = dV[pl.ds(off + g * _L, _L)]
                dV[pl.ds(off + g * _L, _L)] = zf
                b = dB[pl.ds(off + g * _L, _L)]
                dB[pl.ds(off + g * _L, _L)] = zf
                mstage[pl.ds(g * _L, _L)] = val + (cB + b) + a
                dV[pl.ds(off + _G + g * _L, _L)] = zf
                dB[pl.ds(off + _G + g * _L, _L)] = zf

            pltpu.sync_copy(obuf, ob_hbm.at[pl.ds(base * _OB, _OSZ)])
            pltpu.sync_copy(mstage, omb_hbm.at[pl.ds(base, _G)])
            return carry

        lax.fori_loop(0, _CHUNKS, chunk, 0)

    return kern(wf, bf, mb1d)


def kernel(origins, directions, weights, bins, max_bin):
    ob, omb = _sc_sample(weights.reshape(_R * _NC),
                         bins.reshape(_R * _NC), max_bin[:, 0])
    return (origins, directions, ob.reshape(_R, _OB, 1), omb[:, None])


# speedup vs baseline: 14.7461x; 1.3939x over previous
import functools

import jax
import jax.numpy as jnp
from jax import lax
from jax.experimental import pallas as pl
from jax.experimental.pallas import tpu as pltpu
from jax.experimental.pallas import tpu_sc as plsc

_R = 131072
_NC = 64
_NB = 129
_NK = _NC + 1
_OB = _NC + _NB - 1
_ROWS = 194
_PAD = 0.01
_EPS = 1e-5
_NCORES = 2
_NSUB = 16
_NW = _NCORES * _NSUB
_L = 16
_NG = 4
_G = _NG * _L
_RPW = _R // _NW
_CHUNKS = _RPW // _G
_OSZ = _G * _OB


def _sc_sample(wf, bf, mb1d):
    mesh = plsc.VectorSubcoreMesh(core_axis_name="c", subcore_axis_name="s")

    @functools.partial(
        pl.kernel,
        out_type=[
            jax.ShapeDtypeStruct((_R * _OB,), jnp.float32),
            jax.ShapeDtypeStruct((_R,), jnp.float32),
        ],
        mesh=mesh,
        compiler_params=pltpu.CompilerParams(needs_layout_passes=False),
        scratch_types=[
            pltpu.VMEM((_G * _NC,), jnp.float32),
            pltpu.VMEM((_G * _NC,), jnp.float32),
            pltpu.VMEM((_G,), jnp.float32),
            pltpu.VMEM((_ROWS * _G,), jnp.float32),
            pltpu.VMEM((_ROWS * _G,), jnp.float32),
            pltpu.VMEM((_OSZ,), jnp.float32),
            pltpu.VMEM((_G,), jnp.float32),
        ],
    )
    def kern(w_hbm, b_hbm, mb_hbm, ob_hbm, omb_hbm,
             wbuf, bbuf, mbuf, dV, dB, obuf, mstage):
        wid = lax.axis_index("s") * _NCORES + lax.axis_index("c")
        base0 = wid * _RPW
        lane = lax.iota(jnp.int32, _L)
        zf = jnp.zeros((_L,), jnp.float32)
        zi = jnp.zeros((_L,), jnp.int32)
        oi = jnp.ones((_L,), jnp.int32)
        glv = [lane + g * _L for g in range(_NG)]
        grow = [(g * _L) * _NC for g in range(_NG)]
        obase = [(jnp.asarray(g * _L, jnp.int32) + lane) * _OB
                 for g in range(_NG)]

        @plsc.parallel_loop(0, _ROWS)
        def zinit(t):
            for g in range(_NG):
                dV[pl.ds(t * _G + g * _L, _L)] = zf
                dB[pl.ds(t * _G + g * _L, _L)] = zf

        def chunk(ci, carry):
            base = base0 + ci * _G
            pltpu.sync_copy(w_hbm.at[pl.ds(base * _NC, _G * _NC)], wbuf)
            pltpu.sync_copy(b_hbm.at[pl.ds(base * _NC, _G * _NC)], bbuf)
            pltpu.sync_copy(mb_hbm.at[pl.ds(base, _G)], mbuf)

            @plsc.parallel_loop(0, _NC, carry=(zf,) * _NG)
            def S(k, ss):
                return tuple(
                    ss[g] + plsc.load_gather(wbuf, [glv[g] * _NC + k])
                    for g in range(_NG))
            rinv, padc = [], []
            for g in range(_NG):
                Sg = S[g] + (_NC * _PAD)
                padding = jnp.maximum(_EPS - Sg, 0.0)
                rinv.append(1.0 / (Sg + padding))
                padc.append(padding * (1.0 / _NC) + _PAD)

            mvec = [mbuf[pl.ds(g * _L, _L)] for g in range(_NG)]
            e0 = [plsc.load_gather(bbuf, [glv[g] * _NC]) for g in range(_NG)]
            for g in range(_NG):
                plsc.store_scatter(dV, [glv[g]], e0[g])

            def seg_step(k, ek, st, g):
                cs, cdfp, ep, tpf, idxp = st
                wk = plsc.load_gather(wbuf, [glv[g] * _NC + (k - 1)])
                cs = cs + (wk + padc[g]) * rinv[g]
                cdfk = jnp.minimum(cs, 1.0)
                y = cdfk * _NB - 0.5
                iy = y.astype(jnp.int32)
                ck = jnp.maximum(
                    iy + jnp.where(iy.astype(jnp.float32) < y, oi, zi), 0)
                d = cdfk - cdfp
                inv = jnp.where(d > 0, 1.0 / d, 0.0)
                gr = (ek - ep) * inv
                beta = gr * (1.0 / _NB)
                Am = ep + gr * ((0.5 - k) * (1.0 / _NB) - cdfp)
                dv0 = Am + beta * tpf - ep
                idx0 = idxp + _G
                plsc.addupdate_scatter(dV, [idx0], dv0)
                plsc.addupdate_scatter(dB, [idx0], beta)
                tE = ck + k
                tEf = tE.astype(jnp.float32)
                dvE = ek - (Am + beta * (tEf - 1.0))
                idxE = tE * _G + glv[g]
                plsc.addupdate_scatter(dV, [idxE], dvE)
                plsc.addupdate_scatter(dB, [idxE], -beta)
                return (cs, cdfk, ek, tEf, idxE)

            sts0 = tuple((zf, zf, e0[g], zf, glv[g]) for g in range(_NG))

            @plsc.parallel_loop(1, _NC, carry=sts0)
            def sts(k, st):
                return tuple(
                    seg_step(k, plsc.load_gather(bbuf, [glv[g] * _NC + k]),
                             st[g], g)
                    for g in range(_NG))
            for g in range(_NG):
                seg_step(_NC, mvec[g], sts[g], g)

            st0 = tuple((zf, zf, obase[g]) for g in range(_NG))

            @plsc.parallel_loop(0, _OB, carry=st0)
            def st(t, stc):
                off = t * _G
                out = []
                for g in range(_NG):
                    cB, val, oidx = stc[g]
                    a = dV[pl.ds(off + g * _L, _L)]
                    dV[pl.ds(off + g * _L, _L)] = zf
                    b = dB[pl.ds(off + g * _L, _L)]
                    dB[pl.ds(off + g * _L, _L)] = zf
                    cB = cB + b
                    val = val + cB + a
                    plsc.store_scatter(obuf, [oidx], val)
                    out.append((cB, val, oidx + 1))
                return tuple(out)
            off = _OB * _G
            for g in range(_NG):
                cB, val, _ = st[g]
                a = dV[pl.ds(off + g * _L, _L)]
                dV[pl.ds(off + g * _L, _L)] = zf
                b = dB[pl.ds(off + g * _L, _L)]
                dB[pl.ds(off + g * _L, _L)] = zf
                mstage[pl.ds(g * _L, _L)] = val + (cB + b) + a
                dV[pl.ds(off + _G + g * _L, _L)] = zf
                dB[pl.ds(off + _G + g * _L, _L)] = zf

            pltpu.sync_copy(obuf, ob_hbm.at[pl.ds(base * _OB, _OSZ)])
            pltpu.sync_copy(mstage, omb_hbm.at[pl.ds(base, _G)])
            return carry

        lax.fori_loop(0, _CHUNKS, chunk, 0)

    return kern(wf, bf, mb1d)


def kernel(origins, directions, weights, bins, max_bin):
    ob, omb = _sc_sample(weights.reshape(_R * _NC),
                         bins.reshape(_R * _NC), max_bin[:, 0])
    return (origins, directions, ob.reshape(_R, _OB, 1), omb[:, None])
